# Initial kernel scaffold; baseline (speedup 1.0000x reference)
import functools

import jax
import jax.numpy as jnp
from jax import lax
from jax.experimental import pallas as pl
from jax.experimental.pallas import tpu as pltpu
from jax.experimental.pallas import tpu_sc as plsc

N = 10000
E = 320000
D_IN = 128
D_OUT = 64

NC = 2
NS = 16
KA = 80
KC = 80
EPT_A = E // (NC * NS)
EPT_C = E // NS
INIT_ROWS = 1000


def _mesh():
    return plsc.VectorSubcoreMesh(core_axis_name="c", subcore_axis_name="s")


@functools.partial(
    pl.kernel,
    out_type=jax.ShapeDtypeStruct((NC, N), jnp.float32),
    mesh=_mesh(),
    scratch_types=[
        pltpu.VMEM((KA,), jnp.int32),
        pltpu.VMEM((KA,), jnp.float32),
        pltpu.VMEM_SHARED((N,), jnp.float32),
    ],
)
def _sc_degree(dst_hbm, zeros_hbm, degp_hbm, idx_v, ones_v, deg_sh):
    c = lax.axis_index("c")
    s = lax.axis_index("s")

    @pl.when(s == 0)
    def _():
        pltpu.sync_copy(zeros_hbm, deg_sh)

    for i in range(KA // 16):
        ones_v[pl.ds(16 * i, 16)] = jnp.ones((16,), jnp.float32)
    plsc.subcore_barrier()

    tile_base = (c * NS + s) * EPT_A

    def body(i, _):
        base = tile_base + i * KA
        pltpu.sync_copy(dst_hbm.at[pl.ds(base, KA)], idx_v)
        pltpu.sync_copy(ones_v, deg_sh.at[idx_v], add=True)
        return 0

    lax.fori_loop(0, EPT_A // KA, body, 0)
    plsc.subcore_barrier()

    @pl.when(s == 0)
    def _():
        pltpu.sync_copy(deg_sh, degp_hbm.at[c])


@functools.partial(
    pl.kernel,
    out_type=(
        jax.ShapeDtypeStruct((N, D_OUT), jnp.float32),
        jax.ShapeDtypeStruct((N, D_OUT), jnp.float32),
    ),
    mesh=_mesh(),
    scratch_types=[
        pltpu.VMEM((KC,), jnp.int32),
        pltpu.VMEM((KC,), jnp.int32),
        pltpu.VMEM((KC, D_OUT), jnp.float32),
        pltpu.VMEM_SHARED((N, D_OUT), jnp.float32),
        pltpu.SemaphoreType.DMA,
    ],
)
def _sc_accumulate(src_hbm, dst_hbm, ymu_hbm, yls_hbm, accmu_hbm, accls_hbm,
                   sidx_v, didx_v, rows_v, acc_sh, sem):
    c = lax.axis_index("c")
    s = lax.axis_index("s")

    def run(y_hbm, acc_hbm):
        @pl.when(s < N // INIT_ROWS)
        def _():
            sl = pl.ds(s * INIT_ROWS, INIT_ROWS)
            pltpu.sync_copy(y_hbm.at[sl], acc_sh.at[sl])

        plsc.subcore_barrier()
        tile_base = s * EPT_C

        def body(i, _):
            base = tile_base + i * KC
            pltpu.sync_copy(src_hbm.at[pl.ds(base, KC)], sidx_v)
            pltpu.sync_copy(dst_hbm.at[pl.ds(base, KC)], didx_v)
            pltpu.async_copy(y_hbm.at[sidx_v], rows_v, sem).wait()
            pltpu.sync_copy(rows_v, acc_sh.at[didx_v], add=True)
            return 0

        lax.fori_loop(0, EPT_C // KC, body, 0)
        plsc.subcore_barrier()

        @pl.when(s < N // INIT_ROWS)
        def _():
            sl = pl.ds(s * INIT_ROWS, INIT_ROWS)
            pltpu.sync_copy(acc_sh.at[sl], acc_hbm.at[sl])

    @pl.when(c == 0)
    def _():
        run(ymu_hbm, accmu_hbm)

    @pl.when(c == 1)
    def _():
        run(yls_hbm, accls_hbm)


def _tc_prep_body(degp_ref, x_ref, wmu_ref, wls_ref, dinv_ref, ymu_ref, yls_ref):
    deg = degp_ref[0, :] + degp_ref[1, :] + 1.0
    dinv = lax.rsqrt(deg)
    dinv_ref[...] = dinv
    xc = x_ref[...]
    scale = dinv[:, None]
    ymu_ref[...] = jnp.dot(xc, wmu_ref[...], preferred_element_type=jnp.float32) * scale
    yls_ref[...] = jnp.dot(xc, wls_ref[...], preferred_element_type=jnp.float32) * scale


def _tc_prep(degp, x, w_mu, w_ls):
    return pl.pallas_call(
        _tc_prep_body,
        out_shape=(
            jax.ShapeDtypeStruct((N,), jnp.float32),
            jax.ShapeDtypeStruct((N, D_OUT), jnp.float32),
            jax.ShapeDtypeStruct((N, D_OUT), jnp.float32),
        ),
    )(degp, x, w_mu, w_ls)


def _tc_finish_body(accmu_ref, accls_ref, dinv_ref, bmu_ref, bls_ref, mu_ref, ls_ref):
    scale = dinv_ref[...][:, None]
    mu_ref[...] = accmu_ref[...] * scale + bmu_ref[...][None, :]
    ls_ref[...] = accls_ref[...] * scale + bls_ref[...][None, :]


def _tc_finish(accmu, accls, dinv, b_mu, b_ls):
    return pl.pallas_call(
        _tc_finish_body,
        out_shape=(
            jax.ShapeDtypeStruct((N, D_OUT), jnp.float32),
            jax.ShapeDtypeStruct((N, D_OUT), jnp.float32),
        ),
    )(accmu, accls, dinv, b_mu, b_ls)


def kernel(x, edge_index, W_mu, b_mu, W_logstd, b_logstd):
    ei = edge_index.astype(jnp.int32)
    src = ei[0]
    dst = ei[1]
    zeros = jnp.zeros((N,), jnp.float32)
    degp = _sc_degree(dst, zeros)
    dinv, y_mu, y_ls = _tc_prep(degp, x, W_mu, W_logstd)
    acc_mu, acc_ls = _sc_accumulate(src, dst, y_mu, y_ls)
    mu, logstd = _tc_finish(acc_mu, acc_ls, dinv, b_mu, b_logstd)
    return (mu, logstd)


# speedup vs baseline: 22.9603x; 22.9603x over previous
import functools

import jax
import jax.numpy as jnp
from jax import lax
from jax.experimental import pallas as pl
from jax.experimental.pallas import tpu as pltpu
from jax.experimental.pallas import tpu_sc as plsc

N = 10000
E = 320000
D_IN = 128
D_OUT = 64
D2 = 2 * D_OUT

NC = 2
NS = 16
KA = 80
KC = 80
EPT_A = E // (NC * NS)
EPT_C = E // (NC * NS)
INIT_ROWS = 1000


def _mesh():
    return plsc.VectorSubcoreMesh(core_axis_name="c", subcore_axis_name="s")


@functools.partial(
    pl.kernel,
    out_type=jax.ShapeDtypeStruct((NC, N), jnp.float32),
    mesh=_mesh(),
    scratch_types=[
        pltpu.VMEM((KA,), jnp.int32),
        pltpu.VMEM((KA,), jnp.float32),
        pltpu.VMEM_SHARED((N,), jnp.float32),
    ],
)
def _sc_degree(dst_hbm, zeros_hbm, degp_hbm, idx_v, ones_v, deg_sh):
    c = lax.axis_index("c")
    s = lax.axis_index("s")

    @pl.when(s == 0)
    def _():
        pltpu.sync_copy(zeros_hbm, deg_sh)

    for i in range(KA // 16):
        ones_v[pl.ds(16 * i, 16)] = jnp.ones((16,), jnp.float32)
    plsc.subcore_barrier()

    tile_base = (c * NS + s) * EPT_A

    def body(i, _):
        base = tile_base + i * KA
        pltpu.sync_copy(dst_hbm.at[pl.ds(base, KA)], idx_v)
        pltpu.sync_copy(ones_v, deg_sh.at[idx_v], add=True)
        return 0

    lax.fori_loop(0, EPT_A // KA, body, 0)
    plsc.subcore_barrier()

    @pl.when(s == 0)
    def _():
        pltpu.sync_copy(deg_sh, degp_hbm.at[c])


@functools.partial(
    pl.kernel,
    out_type=(
        jax.ShapeDtypeStruct((N, D2), jnp.float32),
        jax.ShapeDtypeStruct((N, D2), jnp.float32),
    ),
    mesh=_mesh(),
    scratch_types=[
        pltpu.VMEM((KC,), jnp.int32),
        pltpu.VMEM((KC,), jnp.int32),
        pltpu.VMEM((KC, D2), jnp.float32),
        pltpu.VMEM_SHARED((N, D2), jnp.float32),
        pltpu.SemaphoreType.DMA,
    ],
)
def _sc_accumulate(src_hbm, dst_hbm, y_hbm, zeros_hbm, acc0_hbm, acc1_hbm,
                   sidx_v, didx_v, rows_v, acc_sh, sem):
    c = lax.axis_index("c")
    s = lax.axis_index("s")

    @pl.when(s < N // INIT_ROWS)
    def _():
        sl = pl.ds(s * INIT_ROWS, INIT_ROWS)

        @pl.when(c == 0)
        def _():
            pltpu.sync_copy(y_hbm.at[sl], acc_sh.at[sl])

        @pl.when(c == 1)
        def _():
            pltpu.sync_copy(zeros_hbm.at[sl], acc_sh.at[sl])

    plsc.subcore_barrier()
    tile_base = (c * NS + s) * EPT_C

    def body(i, _):
        base = tile_base + i * KC
        pltpu.sync_copy(src_hbm.at[pl.ds(base, KC)], sidx_v)
        pltpu.sync_copy(dst_hbm.at[pl.ds(base, KC)], didx_v)
        pltpu.async_copy(y_hbm.at[sidx_v], rows_v, sem).wait()
        pltpu.sync_copy(rows_v, acc_sh.at[didx_v], add=True)
        return 0

    lax.fori_loop(0, EPT_C // KC, body, 0)
    plsc.subcore_barrier()

    @pl.when(s < N // INIT_ROWS)
    def _():
        sl = pl.ds(s * INIT_ROWS, INIT_ROWS)

        @pl.when(c == 0)
        def _():
            pltpu.sync_copy(acc_sh.at[sl], acc0_hbm.at[sl])

        @pl.when(c == 1)
        def _():
            pltpu.sync_copy(acc_sh.at[sl], acc1_hbm.at[sl])


def _tc_prep_body(degp_ref, x_ref, w_ref, dinv_ref, y_ref):
    deg = degp_ref[0, :] + degp_ref[1, :] + 1.0
    dinv = lax.rsqrt(deg)
    dinv_ref[...] = dinv
    y_ref[...] = jnp.dot(x_ref[...], w_ref[...],
                         preferred_element_type=jnp.float32) * dinv[:, None]


def _tc_prep(degp, x, w_cat):
    return pl.pallas_call(
        _tc_prep_body,
        out_shape=(
            jax.ShapeDtypeStruct((N,), jnp.float32),
            jax.ShapeDtypeStruct((N, D2), jnp.float32),
        ),
    )(degp, x, w_cat)


def _tc_finish_body(acc0_ref, acc1_ref, dinv_ref, bmu_ref, bls_ref, mu_ref, ls_ref):
    scale = dinv_ref[...][:, None]
    acc = (acc0_ref[...] + acc1_ref[...]) * scale
    mu_ref[...] = acc[:, :D_OUT] + bmu_ref[...][None, :]
    ls_ref[...] = acc[:, D_OUT:] + bls_ref[...][None, :]


def _tc_finish(acc0, acc1, dinv, b_mu, b_ls):
    return pl.pallas_call(
        _tc_finish_body,
        out_shape=(
            jax.ShapeDtypeStruct((N, D_OUT), jnp.float32),
            jax.ShapeDtypeStruct((N, D_OUT), jnp.float32),
        ),
    )(acc0, acc1, dinv, b_mu, b_ls)


def kernel(x, edge_index, W_mu, b_mu, W_logstd, b_logstd):
    ei = edge_index.astype(jnp.int32)
    src = ei[0]
    dst = ei[1]
    zeros1 = jnp.zeros((N,), jnp.float32)
    zeros2 = jnp.zeros((N, D2), jnp.float32)
    w_cat = jnp.concatenate([W_mu, W_logstd], axis=1)
    degp = _sc_degree(dst, zeros1)
    dinv, y = _tc_prep(degp, x, w_cat)
    acc0, acc1 = _sc_accumulate(src, dst, y, zeros2)
    mu, logstd = _tc_finish(acc0, acc1, dinv, b_mu, b_logstd)
    return (mu, logstd)
